# baseline (speedup 1.0000x reference)
import numpy as np

import jax
import jax.numpy as jnp
from jax import lax
from jax.experimental import pallas as pl
from jax.experimental.pallas import tpu as pltpu
from jax.experimental.pallas import tpu_sc as plsc

N = 10000
E = 320000
D = 128

NC = 2
NS = 16
L = 16
H = 64

CHUNK = 128
NCH0 = 79
NCH1 = 79
NCHT = NCH0 + NCH1
NCH0A = NCH0 + (-NCH0 % 8)
NCH1A = NCH1 + (-NCH1 % 8)
EP = NS * NCHT * CHUNK

NPAD = 10240
ROWS_PER_TILE = NPAD // NS

_PERM = np.zeros(D, np.int32)
for _k in range(D // 32):
    for _i in range(16):
        _PERM[32 * _k + _i] = 32 * _k + 2 * _i
        _PERM[32 * _k + 16 + _i] = 32 * _k + 2 * _i + 1


def _sc_spmm_body(x_hbm, src_hbm, dst_hbm, w_hbm, out_hbm,
                  src_v, dst_v, w_v, raw, msgs_f, agg_sh,
                  gsem0, gsem1, ssem0, ssem1, isem0, isem1):
    cid = lax.axis_index("c")
    sid = lax.axis_index("s")
    base = sid * ROWS_PER_TILE

    off = jnp.where(cid == 0, 0, NCH0)
    nch = jnp.where(cid == 0, NCH0, NCH1)
    npairs = jnp.where(cid == 0, (NCH0 - 1) // 2, (NCH1 - 1) // 2)

    zvec = jnp.zeros((L,), jnp.float32)

    def zero_row(r, c):
        for m in range(D // L):
            msgs_f[0, r, pl.ds(m * L, L)] = zvec
        return c

    lax.fori_loop(0, H, zero_row, 0)
    for i in range(ROWS_PER_TILE // H):
        pltpu.sync_copy(msgs_f.at[0], agg_sh.at[pl.ds(base + i * H, H)])

    @pl.when(cid == 0)
    def _():
        pltpu.sync_copy(dst_hbm.at[sid, pl.ds(0, 2 * NCH0A)],
                        dst_v.at[pl.ds(0, 2 * NCH0A)])

    @pl.when(cid == 1)
    def _():
        pltpu.sync_copy(dst_hbm.at[sid, pl.ds(2 * NCH0A, 2 * NCH1A)],
                        dst_v.at[pl.ds(0, 2 * NCH1A)])

    plsc.subcore_barrier()

    gsems = (gsem0, gsem1)
    ssems = (ssem0, ssem1)
    isems = (isem0, isem1)
    dnums = lax.GatherDimensionNumbers(
        offset_dims=(), collapsed_slice_dims=(0,), start_index_map=(0,))

    def start_idx(jl, b):
        pltpu.async_copy(src_hbm.at[sid, off + jl], src_v.at[b], isems[b])
        pltpu.async_copy(w_hbm.at[sid, off + jl], w_v.at[b], isems[b])

    def wait_idx(b):
        pltpu.make_async_copy(src_hbm.at[sid, 0], src_v.at[b],
                              isems[b]).wait()
        pltpu.make_async_copy(w_hbm.at[sid, 0], w_v.at[b],
                              isems[b]).wait()

    def start_gather(b):
        pltpu.async_copy(x_hbm.at[src_v.at[b]], raw.at[b], gsems[b])

    def wait_gather(b):
        pltpu.make_async_copy(x_hbm.at[src_v.at[b]], raw.at[b],
                              gsems[b]).wait()

    def start_scatter(jl, h):
        pltpu.async_copy(msgs_f.at[h], agg_sh.at[dst_v.at[2 * jl + h]],
                         ssems[h], add=True)

    def wait_scatter(h):
        pltpu.make_async_copy(msgs_f.at[h], agg_sh.at[dst_v.at[0]],
                              ssems[h]).wait()

    def compute_half(b, h):
        def group_body(g, c):
            wgrp = w_v[b, pl.ds((4 * h + g) * L, L)]
            for ei in range(L):
                wv = lax.gather(wgrp, jnp.full((L, 1), ei, jnp.int32),
                                dnums, (1,),
                                mode=lax.GatherScatterMode.PROMISE_IN_BOUNDS)
                e = (4 * h + g) * L + ei
                el = g * L + ei
                for k in range(D // 32):
                    rw = raw[b, e, pl.ds(k * L, L)]
                    va = lax.bitcast_convert_type(rw << 16, jnp.float32)
                    vb = lax.bitcast_convert_type(
                        rw & jnp.int32(-65536), jnp.float32)
                    msgs_f[h, el, pl.ds(k * 32, L)] = va * wv
                    msgs_f[h, el, pl.ds(k * 32 + L, L)] = vb * wv
            return c

        lax.fori_loop(0, H // L, group_body, 0)

    start_idx(0, 0)
    wait_idx(0)
    start_gather(0)
    start_idx(1, 1)

    def pair_body(i, c):
        ja = 2 * i
        wait_gather(0)
        wait_idx(1)
        start_gather(1)

        @pl.when(i > 0)
        def _():
            wait_scatter(0)

        compute_half(0, 0)
        start_scatter(ja, 0)

        @pl.when(i > 0)
        def _():
            wait_scatter(1)

        compute_half(0, 1)
        start_scatter(ja, 1)
        start_idx(ja + 2, 0)
        wait_gather(1)
        wait_idx(0)
        start_gather(0)
        wait_scatter(0)
        compute_half(1, 0)
        start_scatter(ja + 1, 0)
        wait_scatter(1)
        compute_half(1, 1)
        start_scatter(ja + 1, 1)
        start_idx(ja + 3, 1)
        return c

    lax.fori_loop(0, npairs, pair_body, 0)

    wait_gather(0)
    wait_idx(1)
    wait_scatter(0)
    compute_half(0, 0)
    start_scatter(nch - 1, 0)
    wait_scatter(1)
    compute_half(0, 1)
    start_scatter(nch - 1, 1)
    wait_scatter(0)
    wait_scatter(1)
    plsc.subcore_barrier()

    pltpu.sync_copy(agg_sh.at[pl.ds(base, ROWS_PER_TILE)],
                    out_hbm.at[cid, pl.ds(base, ROWS_PER_TILE)])


def _sc_spmm(x_i32, src_p, dst_p, w_p):
    mesh = plsc.VectorSubcoreMesh(
        core_axis_name="c", subcore_axis_name="s", num_cores=NC,
        num_subcores=NS)
    fn = pl.kernel(
        _sc_spmm_body,
        out_type=jax.ShapeDtypeStruct((NC, NPAD, D), jnp.float32),
        mesh=mesh,
        compiler_params=pltpu.CompilerParams(use_tc_tiling_on_sc=False),
        scratch_types=[
            pltpu.VMEM((2, CHUNK), jnp.int32),
            pltpu.VMEM((2 * NCH0A, H), jnp.int32),
            pltpu.VMEM((2, CHUNK), jnp.float32),
            pltpu.VMEM((2, CHUNK, D // 2), jnp.int32),
            pltpu.VMEM((2, H, D), jnp.float32),
            pltpu.VMEM_SHARED((NPAD, D), jnp.float32),
            pltpu.SemaphoreType.DMA,
            pltpu.SemaphoreType.DMA,
            pltpu.SemaphoreType.DMA,
            pltpu.SemaphoreType.DMA,
            pltpu.SemaphoreType.DMA,
            pltpu.SemaphoreType.DMA,
        ],
    )
    return fn(x_i32, src_p, dst_p, w_p)


def _tc_fuse_body(p_ref, y_ref, w_ref, b_ref, o_ref):
    z = p_ref[0] + p_ref[1]
    acc = jnp.dot(z, w_ref[...], preferred_element_type=jnp.float32)
    o_ref[...] = jnp.maximum(acc + b_ref[...] + y_ref[...], 0.0)


def _tc_fuse(partials, y, W, b):
    blk = 1000
    grid = (N // blk,)
    return pl.pallas_call(
        _tc_fuse_body,
        out_shape=jax.ShapeDtypeStruct((N, D), jnp.float32),
        grid=grid,
        in_specs=[
            pl.BlockSpec((NC, blk, D), lambda i: (0, i, 0)),
            pl.BlockSpec((blk, D), lambda i: (i, 0)),
            pl.BlockSpec((D, D), lambda i: (0, 0)),
            pl.BlockSpec((1, D), lambda i: (0, 0)),
        ],
        out_specs=pl.BlockSpec((blk, D), lambda i: (i, 0)),
    )(partials, y, W, b)


@jax.jit
def kernel(x, y, edge_index, edge_weight, W, b):
    pad = EP - E
    src_p = jnp.pad(edge_index[0], (0, pad)).reshape(NS, NCHT, CHUNK)
    dst_p = jnp.pad(edge_index[1], (0, pad)).reshape(NS, NCHT, CHUNK)
    w_p = jnp.pad(edge_weight, (0, pad)).reshape(NS, NCHT, CHUNK)
    src_p = jnp.concatenate(
        [src_p, jnp.zeros((NS, 1, CHUNK), jnp.int32)], axis=1)
    w_p = jnp.concatenate(
        [w_p, jnp.zeros((NS, 1, CHUNK), jnp.float32)], axis=1)
    dst_p = jnp.concatenate(
        [dst_p[:, :NCH0], jnp.zeros((NS, NCH0A - NCH0, CHUNK), jnp.int32),
         dst_p[:, NCH0:], jnp.zeros((NS, NCH1A - NCH1, CHUNK), jnp.int32)],
        axis=1).reshape(NS, 2 * (NCH0A + NCH1A), H)
    x_i32 = lax.bitcast_convert_type(
        x.astype(jnp.bfloat16).reshape(N, D // 2, 2), jnp.int32)
    w_perm = W[_PERM, :]
    partials = _sc_spmm(x_i32, src_p, dst_p, w_p)
    return _tc_fuse(partials, y, w_perm, b.reshape(1, D))


# speedup vs baseline: 1.4127x; 1.4127x over previous
import jax
import jax.numpy as jnp
from jax import lax
from jax.experimental import pallas as pl
from jax.experimental.pallas import tpu as pltpu
from jax.experimental.pallas import tpu_sc as plsc

N = 10000
E = 320000
D = 128

NC = 2
NS = 16
L = 16

CHUNK = 128
NCH0 = 123
NCH1 = 35
NCHT = NCH0 + NCH1
NCH0A = NCH0 + (-NCH0 % 8)
NCH1A = NCH1 + (-NCH1 % 8)
EP = NS * NCHT * CHUNK

NPAD = 10112
ROWS_PER_TILE = NPAD // NS


def _sc_spmm_body(x_hbm, src_hbm, dst_hbm, w_hbm, out_hbm,
                  src_v, dst_v, w_v, msgs, agg_sh,
                  gsem0, gsem1, ssem0, ssem1, isem0, isem1):
    cid = lax.axis_index("c")
    sid = lax.axis_index("s")
    base = sid * ROWS_PER_TILE

    off = jnp.where(cid == 0, 0, NCH0)
    nch = jnp.where(cid == 0, NCH0, NCH1)
    npairs = jnp.where(cid == 0, (NCH0 - 1) // 2, (NCH1 - 1) // 2)

    zvec = jnp.zeros((L,), jnp.float32)

    def zero_row(r, c):
        for m in range(D // L):
            msgs[0, r, pl.ds(m * L, L)] = zvec
        return c

    lax.fori_loop(0, CHUNK, zero_row, 0)
    for i in range(ROWS_PER_TILE // CHUNK):
        pltpu.sync_copy(msgs.at[0], agg_sh.at[pl.ds(base + i * CHUNK, CHUNK)])
    _tail = ROWS_PER_TILE % CHUNK
    if _tail:
        pltpu.sync_copy(
            msgs.at[0, pl.ds(0, _tail)],
            agg_sh.at[pl.ds(base + (ROWS_PER_TILE // CHUNK) * CHUNK, _tail)])

    @pl.when(cid == 0)
    def _():
        pltpu.sync_copy(dst_hbm.at[sid, pl.ds(0, NCH0A)],
                        dst_v.at[pl.ds(0, NCH0A)])

    @pl.when(cid == 1)
    def _():
        pltpu.sync_copy(dst_hbm.at[sid, pl.ds(NCH0A, NCH1A)],
                        dst_v.at[pl.ds(0, NCH1A)])

    plsc.subcore_barrier()

    gsems = (gsem0, gsem1)
    ssems = (ssem0, ssem1)
    isems = (isem0, isem1)
    dnums = lax.GatherDimensionNumbers(
        offset_dims=(), collapsed_slice_dims=(0,), start_index_map=(0,))

    def start_idx(jl, b):
        pltpu.async_copy(src_hbm.at[sid, off + jl], src_v.at[b], isems[b])
        pltpu.async_copy(w_hbm.at[sid, off + jl], w_v.at[b], isems[b])

    def wait_idx(b):
        pltpu.make_async_copy(src_hbm.at[sid, 0], src_v.at[b],
                              isems[b]).wait()
        pltpu.make_async_copy(w_hbm.at[sid, 0], w_v.at[b],
                              isems[b]).wait()

    def start_gather(b):
        pltpu.async_copy(x_hbm.at[src_v.at[b]], msgs.at[b], gsems[b])

    def wait_gather(b):
        pltpu.make_async_copy(x_hbm.at[src_v.at[b]], msgs.at[b],
                              gsems[b]).wait()

    def start_scatter(jl, b):
        pltpu.async_copy(msgs.at[b], agg_sh.at[dst_v.at[jl]], ssems[b],
                         add=True)

    def wait_scatter(b):
        pltpu.make_async_copy(msgs.at[b], agg_sh.at[dst_v.at[0]],
                              ssems[b]).wait()

    def compute(b):
        def group_body(g, c):
            wgrp = w_v[b, pl.ds(g * L, L)]
            for ei in range(L):
                wv = lax.gather(wgrp, jnp.full((L, 1), ei, jnp.int32),
                                dnums, (1,),
                                mode=lax.GatherScatterMode.PROMISE_IN_BOUNDS)
                e = g * L + ei
                for k in range(D // L):
                    sl = (b, e, pl.ds(k * L, L))
                    msgs[sl] = msgs[sl] * wv
            return c

        lax.fori_loop(0, CHUNK // L, group_body, 0)

    start_idx(0, 0)
    wait_idx(0)
    start_gather(0)
    start_idx(1, 1)

    def pair_body(i, c):
        ja = 2 * i
        wait_gather(0)

        @pl.when(i > 0)
        def _():
            wait_scatter(1)

        wait_idx(1)
        start_gather(1)
        compute(0)
        start_scatter(ja, 0)
        start_idx(ja + 2, 0)
        wait_gather(1)
        wait_scatter(0)
        wait_idx(0)
        start_gather(0)
        compute(1)
        start_scatter(ja + 1, 1)
        start_idx(ja + 3, 1)
        return c

    lax.fori_loop(0, npairs, pair_body, 0)

    wait_gather(0)
    wait_idx(1)
    wait_scatter(1)
    compute(0)
    start_scatter(nch - 1, 0)
    wait_scatter(0)
    plsc.subcore_barrier()

    pltpu.sync_copy(agg_sh.at[pl.ds(base, ROWS_PER_TILE)],
                    out_hbm.at[cid, pl.ds(base, ROWS_PER_TILE)])


def _sc_spmm(x, src_p, dst_p, w_p):
    mesh = plsc.VectorSubcoreMesh(
        core_axis_name="c", subcore_axis_name="s", num_cores=NC,
        num_subcores=NS)
    fn = pl.kernel(
        _sc_spmm_body,
        out_type=jax.ShapeDtypeStruct((NC, NPAD, D), jnp.float32),
        mesh=mesh,
        scratch_types=[
            pltpu.VMEM((2, CHUNK), jnp.int32),
            pltpu.VMEM((NCH0A, CHUNK), jnp.int32),
            pltpu.VMEM((2, CHUNK), jnp.float32),
            pltpu.VMEM((2, CHUNK, D), jnp.float32),
            pltpu.VMEM_SHARED((NPAD, D), jnp.float32),
            pltpu.SemaphoreType.DMA,
            pltpu.SemaphoreType.DMA,
            pltpu.SemaphoreType.DMA,
            pltpu.SemaphoreType.DMA,
            pltpu.SemaphoreType.DMA,
            pltpu.SemaphoreType.DMA,
        ],
    )
    return fn(x, src_p, dst_p, w_p)


def _tc_fuse_body(p_ref, y_ref, w_ref, b_ref, o_ref):
    z = p_ref[0] + p_ref[1]
    acc = jnp.dot(z, w_ref[...], preferred_element_type=jnp.float32)
    o_ref[...] = jnp.maximum(acc + b_ref[...] + y_ref[...], 0.0)


def _tc_fuse(partials, y, W, b):
    blk = 1000
    grid = (N // blk,)
    return pl.pallas_call(
        _tc_fuse_body,
        out_shape=jax.ShapeDtypeStruct((N, D), jnp.float32),
        grid=grid,
        in_specs=[
            pl.BlockSpec((NC, blk, D), lambda i: (0, i, 0)),
            pl.BlockSpec((blk, D), lambda i: (i, 0)),
            pl.BlockSpec((D, D), lambda i: (0, 0)),
            pl.BlockSpec((1, D), lambda i: (0, 0)),
        ],
        out_specs=pl.BlockSpec((blk, D), lambda i: (i, 0)),
    )(partials, y, W, b)


@jax.jit
def kernel(x, y, edge_index, edge_weight, W, b):
    pad = EP - E
    src_p = jnp.pad(edge_index[0], (0, pad)).reshape(NS, NCHT, CHUNK)
    dst_p = jnp.pad(edge_index[1], (0, pad)).reshape(NS, NCHT, CHUNK)
    w_p = jnp.pad(edge_weight, (0, pad)).reshape(NS, NCHT, CHUNK)
    src_p = jnp.concatenate(
        [src_p, jnp.zeros((NS, 1, CHUNK), jnp.int32)], axis=1)
    w_p = jnp.concatenate(
        [w_p, jnp.zeros((NS, 1, CHUNK), jnp.float32)], axis=1)
    dst_p = jnp.concatenate(
        [dst_p[:, :NCH0], jnp.zeros((NS, NCH0A - NCH0, CHUNK), jnp.int32),
         dst_p[:, NCH0:], jnp.zeros((NS, NCH1A - NCH1, CHUNK), jnp.int32)],
        axis=1)
    partials = _sc_spmm(x, src_p, dst_p, w_p)
    return _tc_fuse(partials, y, W, b.reshape(1, D))


# speedup vs baseline: 1.4145x; 1.0013x over previous
import jax
import jax.numpy as jnp
from jax import lax
from jax.experimental import pallas as pl
from jax.experimental.pallas import tpu as pltpu
from jax.experimental.pallas import tpu_sc as plsc

N = 10000
E = 320000
D = 128

NC = 2
NS = 16
L = 16

CHUNK = 128
NCH0 = 123
NCH1 = 35
NCHT = NCH0 + NCH1
NCH0A = NCH0 + (-NCH0 % 8)
NCH1A = NCH1 + (-NCH1 % 8)
EP = NS * NCHT * CHUNK

NPAD = 10112
ROWS_PER_TILE = NPAD // NS


def _sc_spmm_body(x_hbm, src_hbm, dst_hbm, w_hbm, out_hbm,
                  src_v, dst_v, w_v, msgs, agg_sh,
                  gsem0, gsem1, ssem0, ssem1, isem0, isem1):
    cid = lax.axis_index("c")
    sid = lax.axis_index("s")
    base = sid * ROWS_PER_TILE

    off = jnp.where(cid == 0, 0, NCH0)
    nch = jnp.where(cid == 0, NCH0, NCH1)
    npairs = jnp.where(cid == 0, (NCH0 - 1) // 2, (NCH1 - 1) // 2)

    zvec = jnp.zeros((L,), jnp.float32)

    def zero_row(r, c):
        for m in range(D // L):
            msgs[0, r, pl.ds(m * L, L)] = zvec
        return c

    lax.fori_loop(0, CHUNK, zero_row, 0)
    for i in range(ROWS_PER_TILE // CHUNK):
        pltpu.sync_copy(msgs.at[0], agg_sh.at[pl.ds(base + i * CHUNK, CHUNK)])
    _tail = ROWS_PER_TILE % CHUNK
    if _tail:
        pltpu.sync_copy(
            msgs.at[0, pl.ds(0, _tail)],
            agg_sh.at[pl.ds(base + (ROWS_PER_TILE // CHUNK) * CHUNK, _tail)])

    @pl.when(cid == 0)
    def _():
        pltpu.sync_copy(dst_hbm.at[sid, pl.ds(0, NCH0A)],
                        dst_v.at[pl.ds(0, NCH0A)])

    @pl.when(cid == 1)
    def _():
        pltpu.sync_copy(dst_hbm.at[sid, pl.ds(NCH0A, NCH1A)],
                        dst_v.at[pl.ds(0, NCH1A)])

    plsc.subcore_barrier()

    gsems = (gsem0, gsem1)
    ssems = (ssem0, ssem1)
    isems = (isem0, isem1)
    dnums = lax.GatherDimensionNumbers(
        offset_dims=(), collapsed_slice_dims=(0,), start_index_map=(0,))

    ebase = (sid * NCHT + off) * CHUNK

    def start_idx(jl, b):
        pltpu.async_copy(src_hbm.at[pl.ds(ebase + jl * CHUNK, CHUNK)],
                         src_v.at[b], isems[b])
        pltpu.async_copy(w_hbm.at[pl.ds(ebase + jl * CHUNK, CHUNK)],
                         w_v.at[b], isems[b])

    def wait_idx(b):
        pltpu.make_async_copy(src_hbm.at[pl.ds(0, CHUNK)], src_v.at[b],
                              isems[b]).wait()
        pltpu.make_async_copy(w_hbm.at[pl.ds(0, CHUNK)], w_v.at[b],
                              isems[b]).wait()

    def start_gather(b):
        pltpu.async_copy(x_hbm.at[src_v.at[b]], msgs.at[b], gsems[b])

    def wait_gather(b):
        pltpu.make_async_copy(x_hbm.at[src_v.at[b]], msgs.at[b],
                              gsems[b]).wait()

    def start_scatter(jl, b):
        pltpu.async_copy(msgs.at[b], agg_sh.at[dst_v.at[jl]], ssems[b],
                         add=True)

    def wait_scatter(b):
        pltpu.make_async_copy(msgs.at[b], agg_sh.at[dst_v.at[0]],
                              ssems[b]).wait()

    def compute(b):
        def group_body(g, c):
            wgrp = w_v[b, pl.ds(g * L, L)]
            for ei in range(L):
                wv = lax.gather(wgrp, jnp.full((L, 1), ei, jnp.int32),
                                dnums, (1,),
                                mode=lax.GatherScatterMode.PROMISE_IN_BOUNDS)
                e = g * L + ei
                for k in range(D // L):
                    sl = (b, e, pl.ds(k * L, L))
                    msgs[sl] = msgs[sl] * wv
            return c

        lax.fori_loop(0, CHUNK // L, group_body, 0)

    start_idx(0, 0)
    wait_idx(0)
    start_gather(0)
    start_idx(1, 1)

    def pair_body(i, c):
        ja = 2 * i
        wait_gather(0)

        @pl.when(i > 0)
        def _():
            wait_scatter(1)

        wait_idx(1)
        start_gather(1)
        compute(0)
        start_scatter(ja, 0)
        start_idx(ja + 2, 0)
        wait_gather(1)
        wait_scatter(0)
        wait_idx(0)
        start_gather(0)
        compute(1)
        start_scatter(ja + 1, 1)
        start_idx(ja + 3, 1)
        return c

    lax.fori_loop(0, npairs, pair_body, 0)

    wait_gather(0)
    wait_idx(1)
    wait_scatter(1)
    compute(0)
    start_scatter(nch - 1, 0)
    wait_scatter(0)
    plsc.subcore_barrier()

    pltpu.sync_copy(agg_sh.at[pl.ds(base, ROWS_PER_TILE)],
                    out_hbm.at[cid, pl.ds(base, ROWS_PER_TILE)])


def _sc_spmm(x, src_p, dst_p, w_p):
    mesh = plsc.VectorSubcoreMesh(
        core_axis_name="c", subcore_axis_name="s", num_cores=NC,
        num_subcores=NS)
    fn = pl.kernel(
        _sc_spmm_body,
        out_type=jax.ShapeDtypeStruct((NC, NPAD, D), jnp.float32),
        mesh=mesh,
        scratch_types=[
            pltpu.VMEM((2, CHUNK), jnp.int32),
            pltpu.VMEM((NCH0A, CHUNK), jnp.int32),
            pltpu.VMEM((2, CHUNK), jnp.float32),
            pltpu.VMEM((2, CHUNK, D), jnp.float32),
            pltpu.VMEM_SHARED((NPAD, D), jnp.float32),
            pltpu.SemaphoreType.DMA,
            pltpu.SemaphoreType.DMA,
            pltpu.SemaphoreType.DMA,
            pltpu.SemaphoreType.DMA,
            pltpu.SemaphoreType.DMA,
            pltpu.SemaphoreType.DMA,
        ],
    )
    return fn(x, src_p, dst_p, w_p)


def _tc_fuse_body(p_ref, y_ref, w_ref, b_ref, o_ref):
    z = p_ref[0] + p_ref[1]
    acc = jnp.dot(z, w_ref[...], preferred_element_type=jnp.float32)
    o_ref[...] = jnp.maximum(acc + b_ref[...] + y_ref[...], 0.0)


def _tc_fuse(partials, y, W, b):
    blk = 1000
    grid = (N // blk,)
    return pl.pallas_call(
        _tc_fuse_body,
        out_shape=jax.ShapeDtypeStruct((N, D), jnp.float32),
        grid=grid,
        in_specs=[
            pl.BlockSpec((NC, blk, D), lambda i: (0, i, 0)),
            pl.BlockSpec((blk, D), lambda i: (i, 0)),
            pl.BlockSpec((D, D), lambda i: (0, 0)),
            pl.BlockSpec((1, D), lambda i: (0, 0)),
        ],
        out_specs=pl.BlockSpec((blk, D), lambda i: (i, 0)),
    )(partials, y, W, b)


@jax.jit
def kernel(x, y, edge_index, edge_weight, W, b):
    src_f = jnp.pad(edge_index[0], (0, EP + CHUNK - E))
    w_f = jnp.pad(edge_weight, (0, EP + CHUNK - E))
    dst_p = jnp.pad(edge_index[1], (0, EP - E)).reshape(NS, NCHT, CHUNK)
    dst_p = jnp.concatenate(
        [dst_p[:, :NCH0], jnp.zeros((NS, NCH0A - NCH0, CHUNK), jnp.int32),
         dst_p[:, NCH0:], jnp.zeros((NS, NCH1A - NCH1, CHUNK), jnp.int32)],
        axis=1)
    partials = _sc_spmm(x, src_f, dst_p, w_f)
    return _tc_fuse(partials, y, W, b.reshape(1, D))


# speedup vs baseline: 1.4400x; 1.0180x over previous
import jax
import jax.numpy as jnp
from jax import lax
from jax.experimental import pallas as pl
from jax.experimental.pallas import tpu as pltpu
from jax.experimental.pallas import tpu_sc as plsc

N = 10000
E = 320000
D = 128

NC = 2
NS = 16
L = 16

CHUNK = 128
NCH0 = 125
NCH1 = 33
NCHT = NCH0 + NCH1
NCH0A = NCH0 + (-NCH0 % 8)
NCH1A = NCH1 + (-NCH1 % 8)
EP = NS * NCHT * CHUNK

NPAD = 10112
ROWS_PER_TILE = NPAD // NS


def _sc_spmm_body(x_hbm, src_hbm, dst_hbm, w_hbm, out_hbm,
                  src_v, dst_v, w_v, msgs, agg_sh,
                  gsem0, gsem1, ssem0, ssem1, isem0, isem1):
    cid = lax.axis_index("c")
    sid = lax.axis_index("s")
    base = sid * ROWS_PER_TILE

    off = jnp.where(cid == 0, 0, NCH0)
    nch = jnp.where(cid == 0, NCH0, NCH1)
    npairs = jnp.where(cid == 0, (NCH0 - 1) // 2, (NCH1 - 1) // 2)

    zvec = jnp.zeros((L,), jnp.float32)

    def zero_row(r, c):
        for m in range(D // L):
            msgs[0, r, pl.ds(m * L, L)] = zvec
        return c

    lax.fori_loop(0, CHUNK, zero_row, 0)
    for i in range(ROWS_PER_TILE // CHUNK):
        pltpu.sync_copy(msgs.at[0], agg_sh.at[pl.ds(base + i * CHUNK, CHUNK)])
    _tail = ROWS_PER_TILE % CHUNK
    if _tail:
        pltpu.sync_copy(
            msgs.at[0, pl.ds(0, _tail)],
            agg_sh.at[pl.ds(base + (ROWS_PER_TILE // CHUNK) * CHUNK, _tail)])

    @pl.when(cid == 0)
    def _():
        pltpu.sync_copy(dst_hbm.at[sid, pl.ds(0, NCH0A)],
                        dst_v.at[pl.ds(0, NCH0A)])

    @pl.when(cid == 1)
    def _():
        pltpu.sync_copy(dst_hbm.at[sid, pl.ds(NCH0A, NCH1A)],
                        dst_v.at[pl.ds(0, NCH1A)])

    plsc.subcore_barrier()

    gsems = (gsem0, gsem1)
    ssems = (ssem0, ssem1)
    isems = (isem0, isem1)
    dnums = lax.GatherDimensionNumbers(
        offset_dims=(), collapsed_slice_dims=(0,), start_index_map=(0,))

    ebase = (sid * NCHT + off) * CHUNK

    def start_idx(jl, b):
        pltpu.async_copy(src_hbm.at[pl.ds(ebase + jl * CHUNK, CHUNK)],
                         src_v.at[b], isems[b])
        pltpu.async_copy(w_hbm.at[pl.ds(ebase + jl * CHUNK, CHUNK)],
                         w_v.at[b], isems[b])

    def wait_idx(b):
        pltpu.make_async_copy(src_hbm.at[pl.ds(0, CHUNK)], src_v.at[b],
                              isems[b]).wait()
        pltpu.make_async_copy(w_hbm.at[pl.ds(0, CHUNK)], w_v.at[b],
                              isems[b]).wait()

    def start_gather(b):
        pltpu.async_copy(x_hbm.at[src_v.at[b]], msgs.at[b], gsems[b])

    def wait_gather(b):
        pltpu.make_async_copy(x_hbm.at[src_v.at[b]], msgs.at[b],
                              gsems[b]).wait()

    def start_scatter(jl, b):
        pltpu.async_copy(msgs.at[b], agg_sh.at[dst_v.at[jl]], ssems[b],
                         add=True)

    def wait_scatter(b):
        pltpu.make_async_copy(msgs.at[b], agg_sh.at[dst_v.at[0]],
                              ssems[b]).wait()

    def compute(b):
        def group_body(g, c):
            wgrp = w_v[b, pl.ds(g * L, L)]
            for ei in range(L):
                wv = lax.gather(wgrp, jnp.full((L, 1), ei, jnp.int32),
                                dnums, (1,),
                                mode=lax.GatherScatterMode.PROMISE_IN_BOUNDS)
                e = g * L + ei
                for k in range(D // L):
                    sl = (b, e, pl.ds(k * L, L))
                    msgs[sl] = msgs[sl] * wv
            return c

        lax.fori_loop(0, CHUNK // L, group_body, 0)

    start_idx(0, 0)
    wait_idx(0)
    start_gather(0)
    start_idx(1, 1)

    def pair_body(i, c):
        ja = 2 * i
        wait_gather(0)

        @pl.when(i > 0)
        def _():
            wait_scatter(1)

        wait_idx(1)
        start_gather(1)
        compute(0)
        start_scatter(ja, 0)
        start_idx(ja + 2, 0)
        wait_gather(1)
        wait_scatter(0)
        wait_idx(0)
        start_gather(0)
        compute(1)
        start_scatter(ja + 1, 1)
        start_idx(ja + 3, 1)
        return c

    lax.fori_loop(0, npairs, pair_body, 0)

    wait_gather(0)
    wait_idx(1)
    wait_scatter(1)
    compute(0)
    start_scatter(nch - 1, 0)
    wait_scatter(0)
    plsc.subcore_barrier()

    pltpu.sync_copy(agg_sh.at[pl.ds(base, ROWS_PER_TILE)],
                    out_hbm.at[cid, pl.ds(base, ROWS_PER_TILE)])


def _sc_spmm(x, src_p, dst_p, w_p):
    mesh = plsc.VectorSubcoreMesh(
        core_axis_name="c", subcore_axis_name="s", num_cores=NC,
        num_subcores=NS)
    fn = pl.kernel(
        _sc_spmm_body,
        out_type=jax.ShapeDtypeStruct((NC, NPAD, D), jnp.float32),
        mesh=mesh,
        scratch_types=[
            pltpu.VMEM((2, CHUNK), jnp.int32),
            pltpu.VMEM((NCH0A, CHUNK), jnp.int32),
            pltpu.VMEM((2, CHUNK), jnp.float32),
            pltpu.VMEM((2, CHUNK, D), jnp.float32),
            pltpu.VMEM_SHARED((NPAD, D), jnp.float32),
            pltpu.SemaphoreType.DMA,
            pltpu.SemaphoreType.DMA,
            pltpu.SemaphoreType.DMA,
            pltpu.SemaphoreType.DMA,
            pltpu.SemaphoreType.DMA,
            pltpu.SemaphoreType.DMA,
        ],
    )
    return fn(x, src_p, dst_p, w_p)


def _tc_fuse_body(p_ref, y_ref, w_ref, b_ref, o_ref):
    z = p_ref[0] + p_ref[1]
    acc = jnp.dot(z, w_ref[...], preferred_element_type=jnp.float32)
    o_ref[...] = jnp.maximum(acc + b_ref[...] + y_ref[...], 0.0)


def _tc_fuse(partials, y, W, b):
    blk = 1000
    grid = (N // blk,)
    return pl.pallas_call(
        _tc_fuse_body,
        out_shape=jax.ShapeDtypeStruct((N, D), jnp.float32),
        grid=grid,
        in_specs=[
            pl.BlockSpec((NC, blk, D), lambda i: (0, i, 0)),
            pl.BlockSpec((blk, D), lambda i: (i, 0)),
            pl.BlockSpec((D, D), lambda i: (0, 0)),
            pl.BlockSpec((1, D), lambda i: (0, 0)),
        ],
        out_specs=pl.BlockSpec((blk, D), lambda i: (i, 0)),
    )(partials, y, W, b)


@jax.jit
def kernel(x, y, edge_index, edge_weight, W, b):
    src_f = jnp.pad(edge_index[0], (0, EP + CHUNK - E))
    w_f = jnp.pad(edge_weight, (0, EP + CHUNK - E))
    dst_p = jnp.pad(edge_index[1], (0, EP - E)).reshape(NS, NCHT, CHUNK)
    dst_p = jnp.concatenate(
        [dst_p[:, :NCH0], jnp.zeros((NS, NCH0A - NCH0, CHUNK), jnp.int32),
         dst_p[:, NCH0:], jnp.zeros((NS, NCH1A - NCH1, CHUNK), jnp.int32)],
        axis=1)
    partials = _sc_spmm(x, src_f, dst_p, w_f)
    return _tc_fuse(partials, y, W, b.reshape(1, D))
